# baseline (speedup 1.0000x reference)
import functools

import jax
import jax.numpy as jnp
from jax import lax
from jax.experimental import pallas as pl
from jax.experimental.pallas import tpu as pltpu
from jax.experimental.pallas import tpu_sc as plsc

_ROWS = 128
_N = 32768
_L = 16
_UNROLL = 8
_SC_ROWS = 32
_SPLIT = _ROWS - _SC_ROWS
_ROWS_PER_BLOCK = 32


def _masked_softmax_block(x_ref, m_ref, o_ref):
    x = x_ref[...]
    m = m_ref[...] == 1
    neg = jnp.finfo(x.dtype).min
    z = jnp.where(m, x, neg)
    mx = jnp.max(z, axis=-1, keepdims=True)
    e = jnp.exp(z - mx)
    s = jnp.sum(e, axis=-1, keepdims=True)
    scale = jnp.where(
        mx > neg,
        jnp.asarray(1.0, x.dtype) / jnp.maximum(s, jnp.asarray(1e-30, x.dtype)),
        jnp.zeros((), x.dtype),
    )
    o_ref[...] = e * scale


def _tc_part(x, m):
    cols = x.shape[1]
    spec = pl.BlockSpec((_ROWS_PER_BLOCK, cols), lambda i: (i, 0))
    return pl.pallas_call(
        _masked_softmax_block,
        grid=(_SPLIT // _ROWS_PER_BLOCK,),
        in_specs=[spec, spec],
        out_specs=spec,
        out_shape=jax.ShapeDtypeStruct((_ROWS, cols), x.dtype),
    )(x, m)


def _sc_masked_softmax(x_hbm, m_hbm, o_hbm, x_v, m_v, o_v):
    nc = 2
    wid = lax.axis_index("s") * nc + lax.axis_index("c")
    row = _SPLIT + wid
    neg = jnp.float32(jnp.finfo(jnp.float32).min)

    pltpu.sync_copy(x_hbm.at[row], x_v)
    pltpu.sync_copy(m_hbm.at[row], m_v)

    n_iters = _N // (_L * _UNROLL)

    def body_max(i, run):
        for u in range(_UNROLL):
            off = (i * _UNROLL + u) * _L
            xv = x_v[pl.ds(off, _L)]
            mv = m_v[pl.ds(off, _L)]
            run = jnp.maximum(run, jnp.where(mv == 1, xv, neg))
        return run

    run = lax.fori_loop(0, n_iters, body_max,
                        jnp.full((_L,), neg, jnp.float32))
    mx = run[0]
    for l in range(1, _L):
        mx = jnp.maximum(mx, run[l])

    def body_exp(i, acc):
        for u in range(_UNROLL):
            off = (i * _UNROLL + u) * _L
            xv = x_v[pl.ds(off, _L)]
            mv = m_v[pl.ds(off, _L)]
            e = jnp.exp(jnp.where(mv == 1, xv, neg) - mx)
            o_v[pl.ds(off, _L)] = e
            acc = acc + e
        return acc

    acc = lax.fori_loop(0, n_iters, body_exp, jnp.zeros((_L,), jnp.float32))
    s = acc[0]
    for l in range(1, _L):
        s = s + acc[l]
    s_vec = jnp.full((_L,), s, jnp.float32)
    mx_vec = jnp.full((_L,), mx, jnp.float32)
    scale = jnp.where(mx_vec > neg,
                      jnp.float32(1.0) / jnp.maximum(s_vec, jnp.float32(1e-30)),
                      jnp.float32(0.0))

    def body_scale(i, carry):
        for u in range(_UNROLL):
            off = (i * _UNROLL + u) * _L
            o_v[pl.ds(off, _L)] = o_v[pl.ds(off, _L)] * scale
        return carry

    lax.fori_loop(0, n_iters, body_scale, 0)
    pltpu.sync_copy(o_v, o_hbm.at[wid])


def _sc_part(x, m):
    mesh = plsc.VectorSubcoreMesh(core_axis_name="c", subcore_axis_name="s")
    k = pl.kernel(
        _sc_masked_softmax,
        mesh=mesh,
        out_type=jax.ShapeDtypeStruct((_SC_ROWS, _N), jnp.float32),
        scratch_types=[
            pltpu.VMEM((_N,), jnp.float32),
            pltpu.VMEM((_N,), jnp.int32),
            pltpu.VMEM((_N,), jnp.float32),
        ],
    )
    return k(x, m)


def kernel(inputLayer, mask):
    full = _tc_part(inputLayer, mask)
    sc = _sc_part(inputLayer, mask)
    return lax.dynamic_update_slice(full, sc, (_SPLIT, 0))


# speedup vs baseline: 2.3530x; 2.3530x over previous
import jax
import jax.numpy as jnp
from jax.experimental import pallas as pl

_ROWS_PER_BLOCK = 64


def _masked_softmax_block(x_ref, m_ref, o_ref):
    x = x_ref[...]
    m = m_ref[...] == 1
    neg = jnp.finfo(x.dtype).min
    z = jnp.where(m, x, neg)
    mx = jnp.max(z, axis=-1, keepdims=True)
    e = jnp.exp(z - mx)
    s = jnp.sum(e, axis=-1, keepdims=True)
    scale = jnp.where(
        mx > neg,
        jnp.asarray(1.0, x.dtype) / jnp.maximum(s, jnp.asarray(1e-30, x.dtype)),
        jnp.zeros((), x.dtype),
    )
    o_ref[...] = e * scale


def kernel(inputLayer, mask):
    rows, cols = inputLayer.shape
    spec = pl.BlockSpec((_ROWS_PER_BLOCK, cols), lambda i: (i, 0))
    return pl.pallas_call(
        _masked_softmax_block,
        grid=(rows // _ROWS_PER_BLOCK,),
        in_specs=[spec, spec],
        out_specs=spec,
        out_shape=jax.ShapeDtypeStruct((rows, cols), inputLayer.dtype),
    )(inputLayer, mask)


# speedup vs baseline: 2.3937x; 1.0173x over previous
import jax
import jax.numpy as jnp
from jax.experimental import pallas as pl
from jax.experimental.pallas import tpu as pltpu

_ROWS = 128
_CHUNK = 16
_NBUF = 4
_NCHUNKS = _ROWS // _CHUNK


def _masked_softmax_rows(x, m_raw):
    m = m_raw == 1
    neg = jnp.finfo(x.dtype).min
    z = jnp.where(m, x, neg)
    mx = jnp.max(z, axis=-1, keepdims=True)
    e = jnp.exp(z - mx)
    s = jnp.sum(e, axis=-1, keepdims=True)
    scale = jnp.where(
        mx > neg,
        jnp.asarray(1.0, x.dtype) / jnp.maximum(s, jnp.asarray(1e-30, x.dtype)),
        jnp.zeros((), x.dtype),
    )
    return e * scale


def _pipelined_kernel(x_hbm, m_hbm, o_hbm, xb, mb, ob, in_sem, out_sem):
    def in_copies(i, slot):
        rows = pl.ds(i * _CHUNK, _CHUNK)
        return (
            pltpu.make_async_copy(x_hbm.at[rows], xb.at[slot], in_sem.at[slot, 0]),
            pltpu.make_async_copy(m_hbm.at[rows], mb.at[slot], in_sem.at[slot, 1]),
        )

    def out_copy(i, slot):
        rows = pl.ds(i * _CHUNK, _CHUNK)
        return pltpu.make_async_copy(ob.at[slot], o_hbm.at[rows], out_sem.at[slot])

    for i in range(_NBUF):
        cx, cm = in_copies(i, i)
        cx.start()
        cm.start()

    for i in range(_NCHUNKS):
        slot = i % _NBUF
        cx, cm = in_copies(i, slot)
        cx.wait()
        cm.wait()
        if i >= _NBUF:
            out_copy(i - _NBUF, slot).wait()
        ob[slot] = _masked_softmax_rows(xb[slot], mb[slot])
        out_copy(i, slot).start()
        nxt = i + _NBUF
        if nxt < _NCHUNKS:
            nx, nm = in_copies(nxt, slot)
            nx.start()
            nm.start()

    for i in range(_NCHUNKS - _NBUF, _NCHUNKS):
        out_copy(i, i % _NBUF).wait()


def kernel(inputLayer, mask):
    rows, cols = inputLayer.shape
    any_spec = pl.BlockSpec(memory_space=pl.ANY)
    return pl.pallas_call(
        _pipelined_kernel,
        in_specs=[any_spec, any_spec],
        out_specs=any_spec,
        out_shape=jax.ShapeDtypeStruct((rows, cols), inputLayer.dtype),
        scratch_shapes=[
            pltpu.VMEM((_NBUF, _CHUNK, cols), jnp.float32),
            pltpu.VMEM((_NBUF, _CHUNK, cols), jnp.int32),
            pltpu.VMEM((_NBUF, _CHUNK, cols), jnp.float32),
            pltpu.SemaphoreType.DMA((_NBUF, 2)),
            pltpu.SemaphoreType.DMA((_NBUF,)),
        ],
    )(inputLayer, mask)


# speedup vs baseline: 2.4083x; 1.0061x over previous
import jax
import jax.numpy as jnp
from jax.experimental import pallas as pl
from jax.experimental.pallas import tpu as pltpu

_ROWS = 128
_CHUNK = 8
_NBUF = 8
_NCHUNKS = _ROWS // _CHUNK


def _masked_softmax_rows(x, m_raw):
    m = m_raw == 1
    neg = jnp.finfo(x.dtype).min
    z = jnp.where(m, x, neg)
    mx = jnp.max(z, axis=-1, keepdims=True)
    e = jnp.exp(z - mx)
    s = jnp.sum(e, axis=-1, keepdims=True)
    scale = jnp.where(
        mx > neg,
        jnp.asarray(1.0, x.dtype) / jnp.maximum(s, jnp.asarray(1e-30, x.dtype)),
        jnp.zeros((), x.dtype),
    )
    return e * scale


def _pipelined_kernel(x_hbm, m_hbm, o_hbm, xb, mb, ob, in_sem, out_sem):
    def in_copies(i, slot):
        rows = pl.ds(i * _CHUNK, _CHUNK)
        return (
            pltpu.make_async_copy(x_hbm.at[rows], xb.at[slot], in_sem.at[slot, 0]),
            pltpu.make_async_copy(m_hbm.at[rows], mb.at[slot], in_sem.at[slot, 1]),
        )

    def out_copy(i, slot):
        rows = pl.ds(i * _CHUNK, _CHUNK)
        return pltpu.make_async_copy(ob.at[slot], o_hbm.at[rows], out_sem.at[slot])

    for i in range(_NBUF):
        cx, cm = in_copies(i, i)
        cx.start()
        cm.start()

    for i in range(_NCHUNKS):
        slot = i % _NBUF
        cx, cm = in_copies(i, slot)
        cx.wait()
        cm.wait()
        if i >= _NBUF:
            out_copy(i - _NBUF, slot).wait()
        ob[slot] = _masked_softmax_rows(xb[slot], mb[slot])
        out_copy(i, slot).start()
        nxt = i + _NBUF
        if nxt < _NCHUNKS:
            nx, nm = in_copies(nxt, slot)
            nx.start()
            nm.start()

    for i in range(_NCHUNKS - _NBUF, _NCHUNKS):
        out_copy(i, i % _NBUF).wait()


def kernel(inputLayer, mask):
    rows, cols = inputLayer.shape
    any_spec = pl.BlockSpec(memory_space=pl.ANY)
    return pl.pallas_call(
        _pipelined_kernel,
        in_specs=[any_spec, any_spec],
        out_specs=any_spec,
        out_shape=jax.ShapeDtypeStruct((rows, cols), inputLayer.dtype),
        scratch_shapes=[
            pltpu.VMEM((_NBUF, _CHUNK, cols), jnp.float32),
            pltpu.VMEM((_NBUF, _CHUNK, cols), jnp.int32),
            pltpu.VMEM((_NBUF, _CHUNK, cols), jnp.float32),
            pltpu.SemaphoreType.DMA((_NBUF, 2)),
            pltpu.SemaphoreType.DMA((_NBUF,)),
        ],
    )(inputLayer, mask)
